# baseline (speedup 1.0000x reference)
import functools

import jax
import jax.numpy as jnp
from jax import lax
from jax.experimental import pallas as pl
from jax.experimental.pallas import tpu as pltpu
from jax.experimental.pallas import tpu_sc as plsc

N = 25000
D = 128
H = 128
G3 = 3 * H

NC, NS = 2, 16
NW = NC * NS
QS = 6272
GARBAGE = QS
ACC_ROWS = 6400
ZPT = ACC_ROWS // NS
CPT = 392
CHUNK = 64
PCH = 200
PSECT = 40
E_PAD = NW * PCH * CHUNK
CAPCH = 60
CAPB = CAPCH * CHUNK
NBUF = 4

ROWB = 1000
GRID = N // ROWB


def _dgT(x, w):
    return lax.dot_general(x, w, (((1,), (1,)), ((), ())),
                           preferred_element_type=jnp.float32)


_sc_mesh = plsc.VectorSubcoreMesh(core_axis_name="c", subcore_axis_name="s",
                                  num_cores=NC, num_subcores=NS)


@functools.partial(
    pl.kernel,
    out_type=[jax.ShapeDtypeStruct((4, NW, CAPB), jnp.int32),
              jax.ShapeDtypeStruct((4, NW, CAPB), jnp.int32)],
    mesh=_sc_mesh,
    compiler_params=pltpu.CompilerParams(needs_layout_passes=False),
    scratch_types=[
        pltpu.VMEM((PSECT, CHUNK), jnp.int32),
        pltpu.VMEM((PSECT, CHUNK), jnp.int32),
        [pltpu.VMEM((CAPB,), jnp.int32) for _ in range(4)],
        [pltpu.VMEM((CAPB,), jnp.int32) for _ in range(4)],
    ],
)
def _partition(srcj_hbm, dstj_hbm, sp_hbm, lp_hbm,
               sj_v, dj_v, bsrc, bloc):
    c = lax.axis_index("c")
    s = lax.axis_index("s")
    w = s * NC + c
    base = w * PCH

    zeros16 = jnp.zeros((16,), jnp.int32)
    garb16 = jnp.full((16,), GARBAGE, jnp.int32)

    def fbody(i, carry):
        for b in range(4):
            bsrc[b][pl.ds(i * 16, 16)] = zeros16
            bloc[b][pl.ds(i * 16, 16)] = garb16
        return carry

    lax.fori_loop(0, CAPB // 16, fbody, 0)

    def sect(k, carry):
        pltpu.sync_copy(srcj_hbm.at[pl.ds(base + k * PSECT, PSECT)], sj_v)
        pltpu.sync_copy(dstj_hbm.at[pl.ds(base + k * PSECT, PSECT)], dj_v)

        def row(rj, cin):
            ns = list(cin)
            for g in range(CHUNK // 16):
                sv = sj_v[rj, pl.ds(g * 16, 16)]
                dv = dj_v[rj, pl.ds(g * 16, 16)]
                bb = ((dv >= QS).astype(jnp.int32)
                      + (dv >= 2 * QS).astype(jnp.int32)
                      + (dv >= 3 * QS).astype(jnp.int32))
                loc = dv - bb * QS
                valid = dv < N
                for b in range(4):
                    m = (bb == b) & valid
                    m32 = m.astype(jnp.int32)
                    pos = ns[b] + lax.cumsum(m32) - m32
                    plsc.store_scatter(bsrc[b], [pos], sv, mask=m)
                    plsc.store_scatter(bloc[b], [pos], loc, mask=m)
                    ns[b] = ns[b] + jnp.sum(m32)
            return tuple(ns)

        return lax.fori_loop(0, PSECT, row, carry)

    zero = jnp.int32(0)
    lax.fori_loop(0, PCH // PSECT, sect, (zero, zero, zero, zero))

    for b in range(4):
        pltpu.sync_copy(bsrc[b], sp_hbm.at[b, w])
        pltpu.sync_copy(bloc[b], lp_hbm.at[b, w])


@functools.partial(
    pl.kernel,
    out_type=jax.ShapeDtypeStruct((N, D), jnp.float32),
    mesh=_sc_mesh,
    scratch_types=[
        pltpu.VMEM((CAPCH, CHUNK), jnp.int32),
        pltpu.VMEM((CAPCH, CHUNK), jnp.int32),
        [pltpu.VMEM((CHUNK, D), jnp.float32)] * NBUF,
        pltpu.VMEM_SHARED((ACC_ROWS, D), jnp.float32),
        [pltpu.SemaphoreType.DMA] * NBUF,
    ],
)
def _scatter_add(msg_hbm, sp_hbm, lp_hbm, out_hbm,
                 src_v, loc_v, bufs, acc, sems):
    c = lax.axis_index("c")
    s = lax.axis_index("s")
    zeros16 = jnp.zeros((16,), jnp.float32)

    for p in range(2):
        b = 2 * c + p

        def zbody(i, carry):
            for l in range(D // 16):
                bufs[0][i, pl.ds(l * 16, 16)] = zeros16
            return carry

        lax.fori_loop(0, CHUNK, zbody, 0)
        zbase = s * ZPT
        for k in range(ZPT // CHUNK):
            pltpu.sync_copy(bufs[0], acc.at[pl.ds(zbase + k * CHUNK, CHUNK)])
        rem = ZPT % CHUNK
        if rem:
            pltpu.sync_copy(
                bufs[0].at[pl.ds(0, rem)],
                acc.at[pl.ds(zbase + (ZPT // CHUNK) * CHUNK, rem)])

        plsc.subcore_barrier()

        for rr in range(2):
            w = 2 * s + rr
            pltpu.sync_copy(sp_hbm.at[b, w], src_v)
            pltpu.sync_copy(lp_hbm.at[b, w], loc_v)

            descs = [pltpu.async_copy(msg_hbm.at[src_v.at[u]], bufs[u],
                                      sems[u])
                     for u in range(NBUF)]

            def ring(t, cin):
                for u in range(NBUF):
                    j = t * NBUF + u
                    descs[u].wait()
                    pltpu.sync_copy(bufs[u], acc.at[loc_v.at[j]], add=True)

                    @pl.when(j + NBUF < CAPCH)
                    def _(u=u, j=j):
                        pltpu.async_copy(msg_hbm.at[src_v.at[j + NBUF]],
                                         bufs[u], sems[u])
                return cin

            lax.fori_loop(0, CAPCH // NBUF, ring, 0)

        plsc.subcore_barrier()

        lo = b * QS
        if p == 0:
            pltpu.sync_copy(acc.at[pl.ds(s * CPT, CPT)],
                            out_hbm.at[pl.ds(lo + s * CPT, CPT)])
        else:
            @pl.when((c == 0) | (s < NS - 1))
            def _():
                pltpu.sync_copy(acc.at[pl.ds(s * CPT, CPT)],
                                out_hbm.at[pl.ds(lo + s * CPT, CPT)])

            tail = N - 3 * QS - (NS - 1) * CPT

            @pl.when((c == 1) & (s == NS - 1))
            def _():
                pltpu.sync_copy(
                    acc.at[pl.ds((NS - 1) * CPT, tail)],
                    out_hbm.at[pl.ds(lo + (NS - 1) * CPT, tail)])

        plsc.subcore_barrier()


def _prep_edges(ei):
    pad = E_PAD - ei.shape[1]
    src = jnp.concatenate([ei[0], jnp.zeros((pad,), jnp.int32)])
    dst = jnp.concatenate([ei[1], jnp.full((pad,), jnp.int32(1 << 30))])
    return (src.reshape(-1, CHUNK).astype(jnp.int32),
            dst.reshape(-1, CHUNK).astype(jnp.int32))


def _full(shape):
    return pl.BlockSpec(shape, lambda i: (0,) * len(shape))


def _rows(width):
    return pl.BlockSpec((ROWB, width), lambda i: (i, 0))


def _combine_body(wih_us, wt_us, wih_su, wt_su, o_us, o_su):
    o_us[...] = jnp.dot(wih_us[...], wt_us[...],
                        preferred_element_type=jnp.float32)
    o_su[...] = jnp.dot(wih_su[...], wt_su[...],
                        preferred_element_type=jnp.float32)


def _combine(wih_us, wt_us, wih_su, wt_su):
    return pl.pallas_call(
        _combine_body,
        out_shape=[jax.ShapeDtypeStruct((G3, D), jnp.float32)] * 2,
    )(wih_us, wt_us, wih_su, wt_su)


def _l1_body(xu, xs, ws_us, ws_su, wc_us, wc_su, b_us, b_su,
             a_us, a_su, gi_spot, gi_user):
    xu_ = xu[...]
    xs_ = xs[...]
    a_us[...] = _dgT(xu_, ws_us[...])
    a_su[...] = _dgT(xs_, ws_su[...])
    gi_spot[...] = _dgT(xs_, wc_us[...]) + b_us[...]
    gi_user[...] = _dgT(xu_, wc_su[...]) + b_su[...]


def _l1(xu, xs, ws_us, ws_su, wc_us, wc_su, b_us, b_su):
    return pl.pallas_call(
        _l1_body,
        grid=(GRID,),
        in_specs=[_rows(D), _rows(D), _full((H, D)), _full((H, D)),
                  _full((G3, D)), _full((G3, D)), _full((1, G3)),
                  _full((1, G3))],
        out_specs=[_rows(D), _rows(D), _rows(G3), _rows(G3)],
        out_shape=[jax.ShapeDtypeStruct((N, D), jnp.float32),
                   jax.ShapeDtypeStruct((N, D), jnp.float32),
                   jax.ShapeDtypeStruct((N, G3), jnp.float32),
                   jax.ShapeDtypeStruct((N, G3), jnp.float32)],
    )(xu, xs, ws_us, ws_su, wc_us, wc_su, b_us, b_su)


def _gru_math(gi, agg, whh, bhh):
    gh = _dgT(agg, whh) + bhh
    r = jax.nn.sigmoid(gi[:, :H] + gh[:, :H])
    z = jax.nn.sigmoid(gi[:, H:2 * H] + gh[:, H:2 * H])
    n = jnp.tanh(gi[:, 2 * H:] + r * gh[:, 2 * H:])
    return jax.nn.relu((1.0 - z) * n + z * agg)


def _gru1_body(gi_s, agg_s, whh_us, bhh_us, gi_u, agg_u, whh_su, bhh_su,
               spot1, user1):
    spot1[...] = _gru_math(gi_s[...], agg_s[...], whh_us[...], bhh_us[...])
    user1[...] = _gru_math(gi_u[...], agg_u[...], whh_su[...], bhh_su[...])


def _gru1(gi_s, agg_s, whh_us, bhh_us, gi_u, agg_u, whh_su, bhh_su):
    return pl.pallas_call(
        _gru1_body,
        grid=(GRID,),
        in_specs=[_rows(G3), _rows(D), _full((G3, H)), _full((1, G3)),
                  _rows(G3), _rows(D), _full((G3, H)), _full((1, G3))],
        out_specs=[_rows(H), _rows(H)],
        out_shape=[jax.ShapeDtypeStruct((N, H), jnp.float32)] * 2,
    )(gi_s, agg_s, whh_us, bhh_us, gi_u, agg_u, whh_su, bhh_su)


def _l2_body(spot1, agg_s, wih_us, whh_us, bih_us, bhh_us,
             user1, agg_u, wih_su, whh_su, bih_su, bhh_su,
             w_spot, b_spot, w_user, b_user,
             user2, spot2, out_user, out_spot):
    gi_s = _dgT(spot1[...], wih_us[...]) + bih_us[...]
    s2 = _gru_math(gi_s, agg_s[...], whh_us[...], bhh_us[...])
    spot2[...] = s2
    out_spot[...] = _dgT(s2, w_spot[...]) + b_spot[0, 0]
    gi_u = _dgT(user1[...], wih_su[...]) + bih_su[...]
    u2 = _gru_math(gi_u, agg_u[...], whh_su[...], bhh_su[...])
    user2[...] = u2
    out_user[...] = _dgT(u2, w_user[...]) + b_user[0, 0]


def _l2(spot1, agg_s, wih_us, whh_us, bih_us, bhh_us,
        user1, agg_u, wih_su, whh_su, bih_su, bhh_su,
        w_spot, b_spot, w_user, b_user):
    return pl.pallas_call(
        _l2_body,
        grid=(GRID,),
        in_specs=[_rows(H), _rows(D), _full((G3, H)), _full((G3, H)),
                  _full((1, G3)), _full((1, G3)),
                  _rows(H), _rows(D), _full((G3, H)), _full((G3, H)),
                  _full((1, G3)), _full((1, G3)),
                  _full((8, H)), _full((1, 1)), _full((8, H)), _full((1, 1))],
        out_specs=[_rows(H), _rows(H), _rows(8), _rows(8)],
        out_shape=[jax.ShapeDtypeStruct((N, H), jnp.float32),
                   jax.ShapeDtypeStruct((N, H), jnp.float32),
                   jax.ShapeDtypeStruct((N, 8), jnp.float32),
                   jax.ShapeDtypeStruct((N, 8), jnp.float32)],
    )(spot1, agg_s, wih_us, whh_us, bih_us, bhh_us,
      user1, agg_u, wih_su, whh_su, bih_su, bhh_su,
      w_spot, b_spot, w_user, b_user)


def kernel(x_user, x_spot, ei_us, ei_su,
           l1_us_Ws, l1_us_Wt, l1_su_Ws, l1_su_Wt,
           l1_us_Wih, l1_us_Whh, l1_us_bih, l1_us_bhh,
           l1_su_Wih, l1_su_Whh, l1_su_bih, l1_su_bhh,
           l2_us_Wih, l2_us_Whh, l2_us_bih, l2_us_bhh,
           l2_su_Wih, l2_su_Whh, l2_su_bih, l2_su_bhh,
           lin_user_W, lin_user_b, lin_spot_W, lin_spot_b):
    r = lambda b: b.reshape(1, -1)

    wc_us, wc_su = _combine(l1_us_Wih, l1_us_Wt, l1_su_Wih, l1_su_Wt)
    a_us, a_su, gi_spot, gi_user = _l1(
        x_user, x_spot, l1_us_Ws, l1_su_Ws, wc_us, wc_su,
        r(l1_us_bih), r(l1_su_bih))

    sj_us, dj_us = _prep_edges(ei_us)
    sj_su, dj_su = _prep_edges(ei_su)

    def part(sj, dj):
        sp, lp = _partition(sj, dj)
        four = lambda a: a.reshape(4, NW, CAPCH, CHUNK)
        return four(sp), four(lp)

    sp_us, lp_us = part(sj_us, dj_us)
    sp_su, lp_su = part(sj_su, dj_su)

    agg_s1 = _scatter_add(a_us, sp_us, lp_us)
    agg_u1 = _scatter_add(a_su, sp_su, lp_su)

    spot1, user1 = _gru1(gi_spot, agg_s1, l1_us_Whh, r(l1_us_bhh),
                         gi_user, agg_u1, l1_su_Whh, r(l1_su_bhh))

    agg_s2 = _scatter_add(user1, sp_us, lp_us)
    agg_u2 = _scatter_add(spot1, sp_su, lp_su)

    pad8 = lambda w: jnp.concatenate([w, jnp.zeros((7, H), jnp.float32)], 0)
    user2, spot2, out_user8, out_spot8 = _l2(
        spot1, agg_s2, l2_us_Wih, l2_us_Whh, r(l2_us_bih), r(l2_us_bhh),
        user1, agg_u2, l2_su_Wih, l2_su_Whh, r(l2_su_bih), r(l2_su_bhh),
        pad8(lin_spot_W), r(lin_spot_b), pad8(lin_user_W), r(lin_user_b))

    return (user2, spot2, out_user8[:, :1], out_spot8[:, :1])


# speedup vs baseline: 7.6868x; 7.6868x over previous
import functools

import jax
import jax.numpy as jnp
from jax import lax
from jax.experimental import pallas as pl
from jax.experimental.pallas import tpu as pltpu
from jax.experimental.pallas import tpu_sc as plsc

N = 25000
D = 128
H = 128
G3 = 3 * H

NC, NS = 2, 16
HALF0 = 12504
GARBAGE = 12504
ACC_ROWS = 12544
ZPT = ACC_ROWS // NS
CPT = 784
CHUNK = 64
NCH = 392
SECT = 56
NSEC = NCH // SECT
NBUF = 2
E_PAD = NS * NCH * CHUNK

ROWB = 1000
GRID = N // ROWB


def _dgT(x, w):
    return lax.dot_general(x, w, (((1,), (1,)), ((), ())),
                           preferred_element_type=jnp.float32)


_sc_mesh = plsc.VectorSubcoreMesh(core_axis_name="c", subcore_axis_name="s",
                                  num_cores=NC, num_subcores=NS)


@functools.partial(
    pl.kernel,
    out_type=jax.ShapeDtypeStruct((N, D), jnp.float32),
    mesh=_sc_mesh,
    scratch_types=[
        pltpu.VMEM((SECT, CHUNK), jnp.int32),
        pltpu.VMEM((SECT, CHUNK), jnp.int32),
        [pltpu.VMEM((CHUNK, D), jnp.float32)] * NBUF,
        pltpu.VMEM_SHARED((ACC_ROWS, D), jnp.float32),
        [pltpu.SemaphoreType.DMA] * NBUF,
    ],
)
def _scatter_add(msg_hbm, srcj_hbm, dstj_hbm, out_hbm,
                 src_v, dst_v, bufs, acc, sems):
    c = lax.axis_index("c")
    s = lax.axis_index("s")
    lo = c * HALF0
    hi = lo + jnp.where(c == 0, HALF0, N - HALF0)

    zeros16 = jnp.zeros((16,), jnp.float32)

    def zbody(i, carry):
        for l in range(D // 16):
            bufs[0][i, pl.ds(l * 16, 16)] = zeros16
        return carry

    lax.fori_loop(0, CHUNK, zbody, 0)
    zbase = s * ZPT
    for k in range(ZPT // CHUNK):
        pltpu.sync_copy(bufs[0], acc.at[pl.ds(zbase + k * CHUNK, CHUNK)])
    rem = ZPT % CHUNK
    if rem:
        pltpu.sync_copy(bufs[0].at[pl.ds(0, rem)],
                        acc.at[pl.ds(zbase + (ZPT // CHUNK) * CHUNK, rem)])

    plsc.subcore_barrier()

    base = s * NCH

    def section(k, carry):
        pltpu.sync_copy(srcj_hbm.at[pl.ds(base + k * SECT, SECT)], src_v)
        pltpu.sync_copy(dstj_hbm.at[pl.ds(base + k * SECT, SECT)], dst_v)

        def lbody(j, cin):
            for l in range(CHUNK // 16):
                d = dst_v[j, pl.ds(l * 16, 16)]
                ok = (d >= lo) & (d < hi)
                dst_v[j, pl.ds(l * 16, 16)] = jnp.where(ok, d - lo, GARBAGE)
            return cin

        lax.fori_loop(0, SECT, lbody, 0)

        descs = [pltpu.async_copy(msg_hbm.at[src_v.at[u]], bufs[u], sems[u])
                 for u in range(NBUF)]

        def ring(t, cin):
            for u in range(NBUF):
                j = t * NBUF + u
                descs[u].wait()
                pltpu.sync_copy(bufs[u], acc.at[dst_v.at[j]], add=True)

                @pl.when(j + NBUF < SECT)
                def _(u=u, j=j):
                    pltpu.async_copy(msg_hbm.at[src_v.at[j + NBUF]],
                                     bufs[u], sems[u])
            return cin

        lax.fori_loop(0, SECT // NBUF, ring, 0)
        return carry

    lax.fori_loop(0, NSEC, section, 0)

    plsc.subcore_barrier()

    for core, half_c in ((0, HALF0), (1, N - HALF0)):
        @pl.when(c == core)
        def _(core=core, half_c=half_c):
            lo_c = core * HALF0
            tail = half_c - (NS - 1) * CPT

            @pl.when(s < NS - 1)
            def _():
                pltpu.sync_copy(acc.at[pl.ds(s * CPT, CPT)],
                                out_hbm.at[pl.ds(lo_c + s * CPT, CPT)])

            @pl.when(s == NS - 1)
            def _():
                pltpu.sync_copy(
                    acc.at[pl.ds((NS - 1) * CPT, tail)],
                    out_hbm.at[pl.ds(lo_c + (NS - 1) * CPT, tail)])


def _prep_edges(ei):
    pad = E_PAD - ei.shape[1]
    src = jnp.concatenate([ei[0], jnp.zeros((pad,), jnp.int32)])
    dst = jnp.concatenate([ei[1], jnp.full((pad,), jnp.int32(1 << 30))])
    return (src.reshape(-1, CHUNK).astype(jnp.int32),
            dst.reshape(-1, CHUNK).astype(jnp.int32))


def _full(shape):
    return pl.BlockSpec(shape, lambda i: (0,) * len(shape))


def _rows(width):
    return pl.BlockSpec((ROWB, width), lambda i: (i, 0))


def _combine_body(wih_us, wt_us, wih_su, wt_su, o_us, o_su):
    o_us[...] = jnp.dot(wih_us[...], wt_us[...],
                        preferred_element_type=jnp.float32)
    o_su[...] = jnp.dot(wih_su[...], wt_su[...],
                        preferred_element_type=jnp.float32)


def _combine(wih_us, wt_us, wih_su, wt_su):
    return pl.pallas_call(
        _combine_body,
        out_shape=[jax.ShapeDtypeStruct((G3, D), jnp.float32)] * 2,
    )(wih_us, wt_us, wih_su, wt_su)


def _l1_body(xu, xs, ws_us, ws_su, wc_us, wc_su, b_us, b_su,
             a_us, a_su, gi_spot, gi_user):
    xu_ = xu[...]
    xs_ = xs[...]
    a_us[...] = _dgT(xu_, ws_us[...])
    a_su[...] = _dgT(xs_, ws_su[...])
    gi_spot[...] = _dgT(xs_, wc_us[...]) + b_us[...]
    gi_user[...] = _dgT(xu_, wc_su[...]) + b_su[...]


def _l1(xu, xs, ws_us, ws_su, wc_us, wc_su, b_us, b_su):
    return pl.pallas_call(
        _l1_body,
        grid=(GRID,),
        in_specs=[_rows(D), _rows(D), _full((H, D)), _full((H, D)),
                  _full((G3, D)), _full((G3, D)), _full((1, G3)),
                  _full((1, G3))],
        out_specs=[_rows(D), _rows(D), _rows(G3), _rows(G3)],
        out_shape=[jax.ShapeDtypeStruct((N, D), jnp.float32),
                   jax.ShapeDtypeStruct((N, D), jnp.float32),
                   jax.ShapeDtypeStruct((N, G3), jnp.float32),
                   jax.ShapeDtypeStruct((N, G3), jnp.float32)],
    )(xu, xs, ws_us, ws_su, wc_us, wc_su, b_us, b_su)


def _gru_math(gi, agg, whh, bhh):
    gh = _dgT(agg, whh) + bhh
    r = jax.nn.sigmoid(gi[:, :H] + gh[:, :H])
    z = jax.nn.sigmoid(gi[:, H:2 * H] + gh[:, H:2 * H])
    n = jnp.tanh(gi[:, 2 * H:] + r * gh[:, 2 * H:])
    return jax.nn.relu((1.0 - z) * n + z * agg)


def _gru1_body(gi_s, agg_s, whh_us, bhh_us, gi_u, agg_u, whh_su, bhh_su,
               spot1, user1):
    spot1[...] = _gru_math(gi_s[...], agg_s[...], whh_us[...], bhh_us[...])
    user1[...] = _gru_math(gi_u[...], agg_u[...], whh_su[...], bhh_su[...])


def _gru1(gi_s, agg_s, whh_us, bhh_us, gi_u, agg_u, whh_su, bhh_su):
    return pl.pallas_call(
        _gru1_body,
        grid=(GRID,),
        in_specs=[_rows(G3), _rows(D), _full((G3, H)), _full((1, G3)),
                  _rows(G3), _rows(D), _full((G3, H)), _full((1, G3))],
        out_specs=[_rows(H), _rows(H)],
        out_shape=[jax.ShapeDtypeStruct((N, H), jnp.float32)] * 2,
    )(gi_s, agg_s, whh_us, bhh_us, gi_u, agg_u, whh_su, bhh_su)


def _l2_body(spot1, agg_s, wih_us, whh_us, bih_us, bhh_us,
             user1, agg_u, wih_su, whh_su, bih_su, bhh_su,
             w_spot, b_spot, w_user, b_user,
             user2, spot2, out_user, out_spot):
    gi_s = _dgT(spot1[...], wih_us[...]) + bih_us[...]
    s2 = _gru_math(gi_s, agg_s[...], whh_us[...], bhh_us[...])
    spot2[...] = s2
    out_spot[...] = _dgT(s2, w_spot[...]) + b_spot[0, 0]
    gi_u = _dgT(user1[...], wih_su[...]) + bih_su[...]
    u2 = _gru_math(gi_u, agg_u[...], whh_su[...], bhh_su[...])
    user2[...] = u2
    out_user[...] = _dgT(u2, w_user[...]) + b_user[0, 0]


def _l2(spot1, agg_s, wih_us, whh_us, bih_us, bhh_us,
        user1, agg_u, wih_su, whh_su, bih_su, bhh_su,
        w_spot, b_spot, w_user, b_user):
    return pl.pallas_call(
        _l2_body,
        grid=(GRID,),
        in_specs=[_rows(H), _rows(D), _full((G3, H)), _full((G3, H)),
                  _full((1, G3)), _full((1, G3)),
                  _rows(H), _rows(D), _full((G3, H)), _full((G3, H)),
                  _full((1, G3)), _full((1, G3)),
                  _full((8, H)), _full((1, 1)), _full((8, H)), _full((1, 1))],
        out_specs=[_rows(H), _rows(H), _rows(8), _rows(8)],
        out_shape=[jax.ShapeDtypeStruct((N, H), jnp.float32),
                   jax.ShapeDtypeStruct((N, H), jnp.float32),
                   jax.ShapeDtypeStruct((N, 8), jnp.float32),
                   jax.ShapeDtypeStruct((N, 8), jnp.float32)],
    )(spot1, agg_s, wih_us, whh_us, bih_us, bhh_us,
      user1, agg_u, wih_su, whh_su, bih_su, bhh_su,
      w_spot, b_spot, w_user, b_user)


def kernel(x_user, x_spot, ei_us, ei_su,
           l1_us_Ws, l1_us_Wt, l1_su_Ws, l1_su_Wt,
           l1_us_Wih, l1_us_Whh, l1_us_bih, l1_us_bhh,
           l1_su_Wih, l1_su_Whh, l1_su_bih, l1_su_bhh,
           l2_us_Wih, l2_us_Whh, l2_us_bih, l2_us_bhh,
           l2_su_Wih, l2_su_Whh, l2_su_bih, l2_su_bhh,
           lin_user_W, lin_user_b, lin_spot_W, lin_spot_b):
    r = lambda b: b.reshape(1, -1)

    wc_us, wc_su = _combine(l1_us_Wih, l1_us_Wt, l1_su_Wih, l1_su_Wt)
    a_us, a_su, gi_spot, gi_user = _l1(
        x_user, x_spot, l1_us_Ws, l1_su_Ws, wc_us, wc_su,
        r(l1_us_bih), r(l1_su_bih))

    sj_us, dj_us = _prep_edges(ei_us)
    sj_su, dj_su = _prep_edges(ei_su)

    agg_s1 = _scatter_add(a_us, sj_us, dj_us)
    agg_u1 = _scatter_add(a_su, sj_su, dj_su)

    spot1, user1 = _gru1(gi_spot, agg_s1, l1_us_Whh, r(l1_us_bhh),
                         gi_user, agg_u1, l1_su_Whh, r(l1_su_bhh))

    agg_s2 = _scatter_add(user1, sj_us, dj_us)
    agg_u2 = _scatter_add(spot1, sj_su, dj_su)

    pad8 = lambda w: jnp.concatenate([w, jnp.zeros((7, H), jnp.float32)], 0)
    user2, spot2, out_user8, out_spot8 = _l2(
        spot1, agg_s2, l2_us_Wih, l2_us_Whh, r(l2_us_bih), r(l2_us_bhh),
        user1, agg_u2, l2_su_Wih, l2_su_Whh, r(l2_su_bih), r(l2_su_bhh),
        pad8(lin_spot_W), r(lin_spot_b), pad8(lin_user_W), r(lin_user_b))

    return (user2, spot2, out_user8[:, :1], out_spot8[:, :1])
